# Initial kernel scaffold; baseline (speedup 1.0000x reference)
import jax
import jax.numpy as jnp
from jax.experimental import pallas as pl


def kernel(x, edge_index, edge_attr, batch, Ws, bs, Wes, gammas, betas):
    raise NotImplementedError("write your pallas kernel here")


# speedup vs baseline: 4.0349x; 4.0349x over previous
import functools

import jax
import jax.numpy as jnp
from jax import lax
from jax.experimental import pallas as pl
from jax.experimental.pallas import tpu as pltpu
from jax.experimental.pallas import tpu_sc as plsc

N = 10000
E = 320000
D = 128
DE = 16
L = 3

NC = 2
NS = 16
NW = NC * NS
CHUNK = 128
EPT = E // NW
NCH = -(-EPT // CHUNK)
EPT_PAD = NCH * CHUNK
ACC_ROWS = 10112
PAD_DST = N
INIT_ROWS = ACC_ROWS // NS

@functools.cache
def _mesh():
    return plsc.VectorSubcoreMesh(core_axis_name="c", subcore_axis_name="s",
                                  num_cores=NC, num_subcores=NS)


def _spmv_body(x_hbm, srcs_hbm, dsts_hbm, zz_hbm, part_hbm,
               rows_v, src_v, dst_v, acc, gsem):
    c = lax.axis_index("c")
    s = lax.axis_index("s")
    wid = c * NS + s

    pltpu.sync_copy(zz_hbm.at[pl.ds(s * INIT_ROWS, INIT_ROWS)],
                    acc.at[pl.ds(s * INIT_ROWS, INIT_ROWS)])
    pltpu.sync_copy(srcs_hbm.at[wid], src_v)
    pltpu.sync_copy(dsts_hbm.at[wid], dst_v)
    plsc.subcore_barrier()

    def chunk(j, carry):
        pltpu.async_copy(x_hbm.at[src_v.at[j]], rows_v, gsem).wait()
        pltpu.sync_copy(rows_v, acc.at[dst_v.at[j]], add=True)
        return carry

    lax.fori_loop(0, NCH, chunk, 0)
    plsc.subcore_barrier()

    pltpu.sync_copy(acc.at[pl.ds(s * INIT_ROWS, INIT_ROWS)],
                    part_hbm.at[c, pl.ds(s * INIT_ROWS, INIT_ROWS)])


@functools.cache
def _sc_spmv():
    return pl.kernel(
        _spmv_body,
        out_type=jax.ShapeDtypeStruct((NC, ACC_ROWS, D), jnp.float32),
        mesh=_mesh(),
        scratch_types=[
            pltpu.VMEM((CHUNK, D), jnp.float32),
            pltpu.VMEM((NCH, CHUNK), jnp.int32),
            pltpu.VMEM((NCH, CHUNK), jnp.int32),
            pltpu.VMEM_SHARED((ACC_ROWS, D), jnp.float32),
            pltpu.SemaphoreType.DMA,
        ],
    )


def _eagg_body(ea_hbm, dsts_hbm, zz2_hbm, eagg_hbm, eab_v, dst_v, acc2, gsem):
    c = lax.axis_index("c")
    s = lax.axis_index("s")
    wid = c * NS + s

    pltpu.sync_copy(zz2_hbm.at[pl.ds(s * INIT_ROWS, INIT_ROWS)],
                    acc2.at[pl.ds(s * INIT_ROWS, INIT_ROWS)])
    pltpu.sync_copy(dsts_hbm.at[wid], dst_v)
    plsc.subcore_barrier()

    def chunk(j, carry):
        pltpu.async_copy(ea_hbm.at[wid, j], eab_v, gsem).wait()
        pltpu.sync_copy(eab_v, acc2.at[dst_v.at[j]], add=True)
        return carry

    lax.fori_loop(0, NCH, chunk, 0)
    plsc.subcore_barrier()

    pltpu.sync_copy(acc2.at[pl.ds(s * INIT_ROWS, INIT_ROWS)],
                    eagg_hbm.at[c, pl.ds(s * INIT_ROWS, INIT_ROWS)])


@functools.cache
def _sc_eagg():
    return pl.kernel(
        _eagg_body,
        out_type=jax.ShapeDtypeStruct((NC, ACC_ROWS, DE), jnp.float32),
        mesh=_mesh(),
        scratch_types=[
            pltpu.VMEM((CHUNK, DE), jnp.float32),
            pltpu.VMEM((NCH, CHUNK), jnp.int32),
            pltpu.VMEM_SHARED((ACC_ROWS, DE), jnp.float32),
            pltpu.SemaphoreType.DMA,
        ],
        compiler_params=pltpu.CompilerParams(use_tc_tiling_on_sc=False),
    )


def _tc_layer(part_ref, eaggp_ref, wes_ref, w_ref, b_ref, g_ref, be_ref,
              out_ref):
    s = part_ref[0, :N] + part_ref[1, :N]
    eagg = eaggp_ref[0, :N] + eaggp_ref[1, :N]
    agg = s + jnp.dot(eagg, wes_ref[...], preferred_element_type=jnp.float32)
    h = jnp.dot(agg, w_ref[...], preferred_element_type=jnp.float32)
    h = h + b_ref[...]
    mean = jnp.mean(h, axis=0, keepdims=True)
    var = jnp.mean((h - mean) ** 2, axis=0, keepdims=True)
    h = (h - mean) * jax.lax.rsqrt(var + 1e-5) * g_ref[...] + be_ref[...]
    out_ref[...] = jnp.maximum(h, 0.0)


_tc_call = pl.pallas_call(
    _tc_layer,
    out_shape=jax.ShapeDtypeStruct((N, D), jnp.float32),
)


def kernel(x, edge_index, edge_attr, batch, Ws, bs, Wes, gammas, betas):
    del batch
    src = edge_index[0].astype(jnp.int32)
    dst = edge_index[1].astype(jnp.int32)

    srcs = jnp.pad(src.reshape(NW, EPT), ((0, 0), (0, EPT_PAD - EPT)))
    srcs = srcs.reshape(NW, NCH, CHUNK)
    dsts = jnp.pad(dst.reshape(NW, EPT), ((0, 0), (0, EPT_PAD - EPT)),
                   constant_values=PAD_DST).reshape(NW, NCH, CHUNK)
    ea = jnp.pad(edge_attr.astype(jnp.float32).reshape(NW, EPT, DE),
                 ((0, 0), (0, EPT_PAD - EPT), (0, 0))).reshape(NW, NCH, CHUNK, DE)
    zz = jnp.zeros((ACC_ROWS, D), jnp.float32)
    zz2 = jnp.zeros((ACC_ROWS, DE), jnp.float32)

    x = x.astype(jnp.float32)
    eaggp = _sc_eagg()(ea, dsts, zz2)
    for l in range(L):
        part = _sc_spmv()(x, srcs, dsts, zz)
        x = _tc_call(part, eaggp, Wes[l], Ws[l], bs[l].reshape(1, D),
                     gammas[l].reshape(1, D), betas[l].reshape(1, D))
    return (x, L)
